# baseline (speedup 1.0000x reference)
import functools

import jax
import jax.numpy as jnp
from jax import lax
from jax.experimental import pallas as pl
from jax.experimental.pallas import tpu as pltpu
from jax.experimental.pallas import tpu_sc as plsc

ACT = 100000
NTILES = 32
ROWS_PT = 3128
LAST_ROWS = ACT - ROWS_PT * (NTILES - 1)
BUF_ROWS = 3136
NEG = -1.0e30
BIG = 1.0e9


def _rne_bf16(v):
    u = lax.bitcast_convert_type(v, jnp.int32)
    r = (u + jnp.int32(0x7FFF) + ((u >> 16) & 1)) & jnp.int32(-65536)
    return lax.bitcast_convert_type(r, jnp.float32)


def _mlp_body(obs_ref, w1_ref, b1_ref, w2_ref, b2_ref, a4_ref, hb_ref):
    obs = obs_ref[...]
    x1 = lax.dot_general(obs, w1_ref[...], (((1,), (1,)), ((), ())),
                         preferred_element_type=jnp.float32)
    x1 = jnp.maximum(x1 + b1_ref[...], 0.0)
    x2 = lax.dot_general(x1, w2_ref[...], (((1,), (1,)), ((), ())),
                         preferred_element_type=jnp.float32)
    x2 = jnp.maximum(x2 + b2_ref[...], 0.0)
    x2 = _rne_bf16(x2)
    rowi = lax.broadcasted_iota(jnp.int32, (4, 16), 0)
    hb = jnp.where(rowi == 0, x2[0, 0],
                   jnp.where(rowi == 1, x2[0, 1],
                             jnp.where(rowi == 2, x2[0, 2], a4_ref[0, 0])))
    hb_ref[...] = hb


def _sc_body(w3_hbm, b3_hbm, hb_hbm, out_hbm, wbuf, bbuf, zbuf, hbuf, ovec,
             sem):
    cid = lax.axis_index("c")
    sid = lax.axis_index("s")
    wid = sid * 2 + cid
    base = wid * ROWS_PT
    last = wid == NTILES - 1
    n_iter = BUF_ROWS // 16

    @pl.when(jnp.logical_not(last))
    def _():
        for c in range(3):
            pltpu.async_copy(w3_hbm.at[pl.ds(c * ACT + base, ROWS_PT)],
                             wbuf.at[pl.ds(c * BUF_ROWS, ROWS_PT)], sem)
        pltpu.async_copy(b3_hbm.at[pl.ds(base, ROWS_PT)],
                         bbuf.at[pl.ds(0, ROWS_PT)], sem)

    @pl.when(last)
    def _():
        for c in range(3):
            pltpu.async_copy(w3_hbm.at[pl.ds(c * ACT + base, LAST_ROWS)],
                             wbuf.at[pl.ds(c * BUF_ROWS, LAST_ROWS)], sem)
        pltpu.async_copy(b3_hbm.at[pl.ds(base, LAST_ROWS)],
                         bbuf.at[pl.ds(0, LAST_ROWS)], sem)

    pltpu.sync_copy(hb_hbm, hbuf)

    h0 = hbuf[pl.ds(0, 16)]
    h1 = hbuf[pl.ds(16, 16)]
    h2 = hbuf[pl.ds(32, 16)]
    a4v = hbuf[pl.ds(48, 16)]

    nrows = jnp.where(last, LAST_ROWS, ROWS_PT)

    lane = lax.iota(jnp.int32, 16)
    lanef = lane.astype(jnp.float32)
    basef = base.astype(jnp.float32)
    basev = basef + lanef

    @pl.when(jnp.logical_not(last))
    def _():
        for c in range(3):
            pltpu.make_async_copy(w3_hbm.at[pl.ds(c * ACT + base, ROWS_PT)],
                                  wbuf.at[pl.ds(c * BUF_ROWS, ROWS_PT)],
                                  sem).wait()
        pltpu.make_async_copy(b3_hbm.at[pl.ds(base, ROWS_PT)],
                              bbuf.at[pl.ds(0, ROWS_PT)], sem).wait()

    @pl.when(last)
    def _():
        for c in range(3):
            pltpu.make_async_copy(w3_hbm.at[pl.ds(c * ACT + base, LAST_ROWS)],
                                  wbuf.at[pl.ds(c * BUF_ROWS, LAST_ROWS)],
                                  sem).wait()
        pltpu.make_async_copy(b3_hbm.at[pl.ds(base, LAST_ROWS)],
                              bbuf.at[pl.ds(0, LAST_ROWS)], sem).wait()

    def _z_at(i):
        o = i * 16
        w0 = wbuf[pl.ds(o, 16)]
        w1 = wbuf[pl.ds(BUF_ROWS + o, 16)]
        w2 = wbuf[pl.ds(2 * BUF_ROWS + o, 16)]
        bv = bbuf[pl.ds(o, 16)]
        return w0 * h0 + w1 * h1 + w2 * h2 + bv

    mv = jnp.full((16,), NEG, jnp.float32)
    for i in range(n_iter):
        z = _z_at(i)
        if i == 0:
            z = jnp.where(base + lane < 2, z * a4v, z)
        z = jnp.where(lane + (16 * i) < nrows, z, NEG)
        zbuf[pl.ds(i * 16, 16)] = z
        mv = jnp.maximum(mv, z)

    m_t = jnp.max(mv)
    mb = jnp.broadcast_to(m_t, (16,))

    sv = jnp.zeros((16,), jnp.float32)
    av = jnp.zeros((16,), jnp.float32)
    iv = jnp.full((16,), BIG, jnp.float32)
    for i in range(n_iter):
        z = zbuf[pl.ds(i * 16, 16)]
        e = jnp.exp(z - mb)
        sv = sv + e
        av = av + z * e
        iv = jnp.minimum(iv, jnp.where(z == mb, basev + float(16 * i), BIG))

    s_t = jnp.sum(sv)
    a_t = jnp.sum(av)
    i_t = jnp.min(iv)

    out = jnp.where(lane == 0, m_t,
                    jnp.where(lane == 1, i_t,
                              jnp.where(lane == 2, s_t,
                                        jnp.where(lane == 3, a_t, 0.0))))
    ovec[...] = out
    pltpu.sync_copy(ovec, out_hbm.at[wid])


def _merge_body(part_ref, a_ref, logp_ref, ent_ref):
    part = part_ref[...]
    m_t = part[:, 0:1]
    idx = part[:, 1:2]
    s_t = part[:, 2:3]
    a_t = part[:, 3:4]
    m_g = jnp.max(m_t)
    w = jnp.exp(m_t - m_g)
    z = jnp.sum(s_t * w)
    za = jnp.sum(a_t * w)
    amax = jnp.min(jnp.where(m_t == m_g, idx, 2.0 * BIG))
    logz = jnp.log(z)
    a_ref[...] = jnp.broadcast_to(amax, (1, 1)).astype(jnp.int32)
    logp_ref[...] = jnp.broadcast_to(-logz, (1, 1))
    ent_ref[...] = jnp.broadcast_to(m_g + logz - za / z, (1, 1))


def _sc_call():
    return functools.partial(
        pl.kernel,
        out_type=jax.ShapeDtypeStruct((NTILES, 16), jnp.float32),
        mesh=plsc.VectorSubcoreMesh(core_axis_name="c", subcore_axis_name="s",
                                    num_cores=2, num_subcores=16),
        scratch_types=[
            pltpu.VMEM((3 * BUF_ROWS,), jnp.float32),
            pltpu.VMEM((BUF_ROWS,), jnp.float32),
            pltpu.VMEM((BUF_ROWS,), jnp.float32),
            pltpu.VMEM((64,), jnp.float32),
            pltpu.VMEM((16,), jnp.float32),
            pltpu.SemaphoreType.DMA,
        ],
        compiler_params=pltpu.CompilerParams(needs_layout_passes=False),
    )


def kernel(obs, W1, b1, W2, b2, W3, b3, a4):
    hb = pl.pallas_call(
        _mlp_body,
        out_shape=jax.ShapeDtypeStruct((4, 16), jnp.float32),
    )(obs, W1, b1.reshape(1, 3), W2, b2.reshape(1, 3), a4.reshape(1, 1))

    w3c = _rne_bf16(W3.T).reshape(-1)
    part = _sc_call()(_sc_body)(w3c, b3, hb.reshape(-1))

    a, logp, ent = pl.pallas_call(
        _merge_body,
        out_shape=(
            jax.ShapeDtypeStruct((1, 1), jnp.int32),
            jax.ShapeDtypeStruct((1, 1), jnp.float32),
            jax.ShapeDtypeStruct((1, 1), jnp.float32),
        ),
    )(part)
    return (a.reshape(()), logp.reshape(1), ent.reshape(1))


# speedup vs baseline: 1.0856x; 1.0856x over previous
import functools

import jax
import jax.numpy as jnp
from jax import lax
from jax.experimental import pallas as pl
from jax.experimental.pallas import tpu as pltpu
from jax.experimental.pallas import tpu_sc as plsc

ACT = 100000
NTILES = 32
ROWS_PT = 3128
LAST_ROWS = ACT - ROWS_PT * (NTILES - 1)
BUF_ROWS = 3136
NEG = -1.0e30
BIG = 1.0e9


def _rne_bf16(v):
    u = lax.bitcast_convert_type(v, jnp.int32)
    r = (u + jnp.int32(0x7FFF) + ((u >> 16) & 1)) & jnp.int32(-65536)
    return lax.bitcast_convert_type(r, jnp.float32)


def _mlp_body(obs_ref, w1_ref, b1_ref, w2_ref, b2_ref, a4_ref, hb_ref):
    obs = obs_ref[...]
    x1 = lax.dot_general(obs, w1_ref[...], (((1,), (1,)), ((), ())),
                         preferred_element_type=jnp.float32)
    x1 = jnp.maximum(x1 + b1_ref[...], 0.0)
    x2 = lax.dot_general(x1, w2_ref[...], (((1,), (1,)), ((), ())),
                         preferred_element_type=jnp.float32)
    x2 = jnp.maximum(x2 + b2_ref[...], 0.0)
    x2 = _rne_bf16(x2)
    rowi = lax.broadcasted_iota(jnp.int32, (4, 16), 0)
    hb = jnp.where(rowi == 0, x2[0, 0],
                   jnp.where(rowi == 1, x2[0, 1],
                             jnp.where(rowi == 2, x2[0, 2], a4_ref[0, 0])))
    hb_ref[...] = hb


def _sc_body(w3_hbm, b3_hbm, hb_hbm, out_hbm, wbuf, bbuf, zbuf, hbuf, ovec,
             sem):
    cid = lax.axis_index("c")
    sid = lax.axis_index("s")
    wid = sid * 2 + cid
    base = wid * ROWS_PT
    last = wid == NTILES - 1
    n_iter = BUF_ROWS // 16

    @pl.when(jnp.logical_not(last))
    def _():
        for c in range(3):
            pltpu.async_copy(w3_hbm.at[pl.ds(c * ACT + base, ROWS_PT)],
                             wbuf.at[pl.ds(c * BUF_ROWS, ROWS_PT)], sem)
        pltpu.async_copy(b3_hbm.at[pl.ds(base, ROWS_PT)],
                         bbuf.at[pl.ds(0, ROWS_PT)], sem)

    @pl.when(last)
    def _():
        for c in range(3):
            pltpu.async_copy(w3_hbm.at[pl.ds(c * ACT + base, LAST_ROWS)],
                             wbuf.at[pl.ds(c * BUF_ROWS, LAST_ROWS)], sem)
        pltpu.async_copy(b3_hbm.at[pl.ds(base, LAST_ROWS)],
                         bbuf.at[pl.ds(0, LAST_ROWS)], sem)

    pltpu.sync_copy(hb_hbm, hbuf)

    h0 = hbuf[pl.ds(0, 16)]
    h1 = hbuf[pl.ds(16, 16)]
    h2 = hbuf[pl.ds(32, 16)]
    a4v = hbuf[pl.ds(48, 16)]

    nrows = jnp.where(last, LAST_ROWS, ROWS_PT)

    lane = lax.iota(jnp.int32, 16)
    lanef = lane.astype(jnp.float32)
    basef = base.astype(jnp.float32)
    basev = basef + lanef

    @pl.when(jnp.logical_not(last))
    def _():
        for c in range(3):
            pltpu.make_async_copy(w3_hbm.at[pl.ds(c * ACT + base, ROWS_PT)],
                                  wbuf.at[pl.ds(c * BUF_ROWS, ROWS_PT)],
                                  sem).wait()
        pltpu.make_async_copy(b3_hbm.at[pl.ds(base, ROWS_PT)],
                              bbuf.at[pl.ds(0, ROWS_PT)], sem).wait()

    @pl.when(last)
    def _():
        for c in range(3):
            pltpu.make_async_copy(w3_hbm.at[pl.ds(c * ACT + base, LAST_ROWS)],
                                  wbuf.at[pl.ds(c * BUF_ROWS, LAST_ROWS)],
                                  sem).wait()
        pltpu.make_async_copy(b3_hbm.at[pl.ds(base, LAST_ROWS)],
                              bbuf.at[pl.ds(0, LAST_ROWS)], sem).wait()

    def _z_at(i):
        o = i * 16
        w0 = wbuf[pl.ds(o, 16)]
        w1 = wbuf[pl.ds(BUF_ROWS + o, 16)]
        w2 = wbuf[pl.ds(2 * BUF_ROWS + o, 16)]
        bv = bbuf[pl.ds(o, 16)]
        return w0 * h0 + w1 * h1 + w2 * h2 + bv

    nfull = nrows // 16

    z0 = _z_at(0)
    z0 = jnp.where(base + lane < 2, z0 * a4v, z0)
    zbuf[pl.ds(0, 16)] = z0

    def _p1(i, mv):
        z = _z_at(i)
        zbuf[pl.ds(i * 16, 16)] = z
        return jnp.maximum(mv, z)

    mv = lax.fori_loop(1, nfull, _p1, z0)

    zt = _z_at(nfull)
    zt = jnp.where(lane < 8, zt, NEG)
    zbuf[pl.ds(nfull * 16, 16)] = zt
    mv = jnp.maximum(mv, zt)

    m_t = jnp.max(mv)
    mb = jnp.broadcast_to(m_t, (16,))

    def _p2(i, carry):
        sv, av, iv = carry
        z = zbuf[pl.ds(i * 16, 16)]
        e = jnp.exp(z - mb)
        sv = sv + e
        av = av + z * e
        growf = basev + (i * 16).astype(jnp.float32)
        iv = jnp.minimum(iv, jnp.where(z == mb, growf, BIG))
        return (sv, av, iv)

    zero = jnp.zeros((16,), jnp.float32)
    sv, av, iv = lax.fori_loop(0, nfull + 1, _p2,
                               (zero, zero, jnp.full((16,), BIG, jnp.float32)))

    s_t = jnp.sum(sv)
    a_t = jnp.sum(av)
    i_t = jnp.min(iv)

    out = jnp.where(lane == 0, m_t,
                    jnp.where(lane == 1, i_t,
                              jnp.where(lane == 2, s_t,
                                        jnp.where(lane == 3, a_t, 0.0))))
    ovec[...] = out
    pltpu.sync_copy(ovec, out_hbm.at[wid])


def _merge_body(part_ref, a_ref, logp_ref, ent_ref):
    part = part_ref[...]
    m_t = part[:, 0:1]
    idx = part[:, 1:2]
    s_t = part[:, 2:3]
    a_t = part[:, 3:4]
    m_g = jnp.max(m_t)
    w = jnp.exp(m_t - m_g)
    z = jnp.sum(s_t * w)
    za = jnp.sum(a_t * w)
    amax = jnp.min(jnp.where(m_t == m_g, idx, 2.0 * BIG))
    logz = jnp.log(z)
    a_ref[...] = jnp.broadcast_to(amax, (1, 1)).astype(jnp.int32)
    logp_ref[...] = jnp.broadcast_to(-logz, (1, 1))
    ent_ref[...] = jnp.broadcast_to(m_g + logz - za / z, (1, 1))


def _sc_call():
    return functools.partial(
        pl.kernel,
        out_type=jax.ShapeDtypeStruct((NTILES, 16), jnp.float32),
        mesh=plsc.VectorSubcoreMesh(core_axis_name="c", subcore_axis_name="s",
                                    num_cores=2, num_subcores=16),
        scratch_types=[
            pltpu.VMEM((3 * BUF_ROWS,), jnp.float32),
            pltpu.VMEM((BUF_ROWS,), jnp.float32),
            pltpu.VMEM((BUF_ROWS,), jnp.float32),
            pltpu.VMEM((64,), jnp.float32),
            pltpu.VMEM((16,), jnp.float32),
            pltpu.SemaphoreType.DMA,
        ],
        compiler_params=pltpu.CompilerParams(needs_layout_passes=False),
    )


def kernel(obs, W1, b1, W2, b2, W3, b3, a4):
    hb = pl.pallas_call(
        _mlp_body,
        out_shape=jax.ShapeDtypeStruct((4, 16), jnp.float32),
    )(obs, W1, b1.reshape(1, 3), W2, b2.reshape(1, 3), a4.reshape(1, 1))

    w3c = _rne_bf16(W3.T).reshape(-1)
    part = _sc_call()(_sc_body)(w3c, b3, hb.reshape(-1))

    a, logp, ent = pl.pallas_call(
        _merge_body,
        out_shape=(
            jax.ShapeDtypeStruct((1, 1), jnp.int32),
            jax.ShapeDtypeStruct((1, 1), jnp.float32),
            jax.ShapeDtypeStruct((1, 1), jnp.float32),
        ),
    )(part)
    return (a.reshape(()), logp.reshape(1), ent.reshape(1))
